# baseline (speedup 1.0000x reference)
import functools

import jax
import jax.numpy as jnp
from jax import lax
from jax.experimental import pallas as pl
from jax.experimental.pallas import tpu as pltpu
from jax.experimental.pallas import tpu_sc as plsc

B = 4096
S = 200
D = 64
H = 256
C = 50
CPAD = 128

NC = 2
NS = 16
NW = NC * NS
BPW = B // NW
SA = 104
SB = S - SA
SP = 256
LANES = 16
DCH = D // LANES


def _sc_bag(X, table):
  mesh = plsc.VectorSubcoreMesh(core_axis_name="c", subcore_axis_name="s")

  @functools.partial(
      pl.kernel,
      out_type=jax.ShapeDtypeStruct((B, D), jnp.float32),
      mesh=mesh,
      compiler_params=pltpu.CompilerParams(use_tc_tiling_on_sc=False),
      scratch_types=[
          pltpu.VMEM((BPW, SP), jnp.int32),
          pltpu.VMEM((4, SA, D), jnp.float32),
          pltpu.VMEM((4, SB, D), jnp.float32),
          pltpu.VMEM((BPW, D), jnp.float32),
          pltpu.SemaphoreType.DMA((4,)),
          pltpu.SemaphoreType.DMA((4,)),
      ],
  )
  def bag(x_hbm, tab_hbm, out_hbm, idx_v, buf_a, buf_b, out_v, sem_a, sem_b):
    w = lax.axis_index("s") * NC + lax.axis_index("c")
    base = w * BPW
    pltpu.sync_copy(x_hbm.at[pl.ds(base, BPW)], idx_v)

    def start_a(b, k):
      pltpu.async_copy(
          tab_hbm.at[idx_v.at[b, pl.ds(0, SA)]], buf_a.at[k], sem_a.at[k]
      )

    def start_b(b, k):
      pltpu.async_copy(
          tab_hbm.at[idx_v.at[b, pl.ds(SA, SB)]], buf_b.at[k], sem_b.at[k]
      )

    def wait_a(k):
      pltpu.make_async_copy(
          tab_hbm.at[idx_v.at[0, pl.ds(0, SA)]], buf_a.at[k], sem_a.at[k]
      ).wait()

    def wait_b(k):
      pltpu.make_async_copy(
          tab_hbm.at[idx_v.at[0, pl.ds(SA, SB)]], buf_b.at[k], sem_b.at[k]
      ).wait()

    def accum(buf, n, accs):
      def body(i, accs):
        r = i * 4
        return tuple(
            accs[c]
            + (
                buf[r, pl.ds(c * LANES, LANES)]
                + buf[r + 1, pl.ds(c * LANES, LANES)]
            )
            + (
                buf[r + 2, pl.ds(c * LANES, LANES)]
                + buf[r + 3, pl.ds(c * LANES, LANES)]
            )
            for c in range(DCH)
        )
      return lax.fori_loop(0, n // 4, body, accs)

    for k in range(4):
      start_a(k, k)
      start_b(k, k)

    @pl.loop(0, BPW, step=4)
    def _(b):
      for k in range(4):
        zeros = tuple(jnp.zeros((LANES,), jnp.float32) for _ in range(DCH))
        wait_a(k)
        acc = accum(buf_a.at[k], SA, zeros)

        @pl.when(b + k + 4 < BPW)
        def _():
          start_a(b + k + 4, k)

        wait_b(k)
        acc = accum(buf_b.at[k], SB, acc)

        @pl.when(b + k + 4 < BPW)
        def _():
          start_b(b + k + 4, k)

        for c in range(DCH):
          out_v[b + k, pl.ds(c * LANES, LANES)] = acc[c] * (1.0 / S)

    pltpu.sync_copy(out_v, out_hbm.at[pl.ds(base, BPW)])

  return bag(X, table)


def _mlp_body(x_ref, w1_ref, b1_ref, w2_ref, b2_ref, o_ref):
  x = x_ref[...]
  h = jnp.maximum(
      jnp.dot(x, w1_ref[...], preferred_element_type=jnp.float32) + b1_ref[...],
      0.0,
  )
  logits = (
      jnp.dot(h, w2_ref[...], preferred_element_type=jnp.float32) + b2_ref[...]
  )
  m = jnp.max(logits, axis=-1, keepdims=True)
  s = logits - m
  lse = jnp.log(jnp.sum(jnp.exp(s), axis=-1, keepdims=True))
  o_ref[...] = s - lse


def _mlp(bag, W1, b1, W2p, b2p):
  BB = 512
  return pl.pallas_call(
      _mlp_body,
      grid=(B // BB,),
      in_specs=[
          pl.BlockSpec((BB, D), lambda i: (i, 0)),
          pl.BlockSpec((D, H), lambda i: (0, 0)),
          pl.BlockSpec((1, H), lambda i: (0, 0)),
          pl.BlockSpec((H, CPAD), lambda i: (0, 0)),
          pl.BlockSpec((1, CPAD), lambda i: (0, 0)),
      ],
      out_specs=pl.BlockSpec((BB, CPAD), lambda i: (i, 0)),
      out_shape=jax.ShapeDtypeStruct((B, CPAD), jnp.float32),
  )(bag, W1, b1, W2p, b2p)


@jax.jit
def kernel(X, table, W1, b1, W2, b2):
  Xp = jnp.pad(X, ((0, 0), (0, SP - S)))
  bag = _sc_bag(Xp, table)
  W2p = jnp.pad(W2, ((0, 0), (0, CPAD - C)))
  b2p = jnp.pad(b2, (0, CPAD - C), constant_values=-1e30).reshape(1, CPAD)
  out = _mlp(bag, W1, b1.reshape(1, H), W2p, b2p)
  return out[:, :C]


# speedup vs baseline: 1.0027x; 1.0027x over previous
import functools

import jax
import jax.numpy as jnp
from jax import lax
from jax.experimental import pallas as pl
from jax.experimental.pallas import tpu as pltpu
from jax.experimental.pallas import tpu_sc as plsc

B = 4096
S = 200
D = 64
H = 256
C = 50
CPAD = 128

NC = 2
NS = 16
NW = NC * NS
BPW = B // NW
SA = 104
SB = S - SA
SP = 256
LANES = 16
DCH = D // LANES


def _sc_bag(X, table):
  mesh = plsc.VectorSubcoreMesh(core_axis_name="c", subcore_axis_name="s")

  @functools.partial(
      pl.kernel,
      out_type=jax.ShapeDtypeStruct((B, D), jnp.float32),
      mesh=mesh,
      compiler_params=pltpu.CompilerParams(use_tc_tiling_on_sc=False),
      scratch_types=[
          pltpu.VMEM((BPW, SP), jnp.int32),
          pltpu.VMEM((4, SA, D), jnp.float32),
          pltpu.VMEM((4, SB, D), jnp.float32),
          pltpu.VMEM((BPW, D), jnp.float32),
          pltpu.SemaphoreType.DMA((4,)),
          pltpu.SemaphoreType.DMA((4,)),
      ],
  )
  def bag(x_hbm, tab_hbm, out_hbm, idx_v, buf_a, buf_b, out_v, sem_a, sem_b):
    w = lax.axis_index("s") * NC + lax.axis_index("c")
    base = w * BPW
    pltpu.sync_copy(x_hbm.at[pl.ds(base, BPW)], idx_v)

    def start_a(b, k):
      pltpu.async_copy(
          tab_hbm.at[idx_v.at[b, pl.ds(0, SA)]], buf_a.at[k], sem_a.at[k]
      )

    def start_b(b, k):
      pltpu.async_copy(
          tab_hbm.at[idx_v.at[b, pl.ds(SA, SB)]], buf_b.at[k], sem_b.at[k]
      )

    def wait_a(k):
      pltpu.make_async_copy(
          tab_hbm.at[idx_v.at[0, pl.ds(0, SA)]], buf_a.at[k], sem_a.at[k]
      ).wait()

    def wait_b(k):
      pltpu.make_async_copy(
          tab_hbm.at[idx_v.at[0, pl.ds(SA, SB)]], buf_b.at[k], sem_b.at[k]
      ).wait()

    def accum(buf, n, accs):
      def body(i, accs):
        r = i * 2
        return tuple(
            accs[c]
            + buf[r, pl.ds(c * LANES, LANES)]
            + buf[r + 1, pl.ds(c * LANES, LANES)]
            for c in range(DCH)
        )
      return lax.fori_loop(0, n // 2, body, accs)

    for k in range(4):
      start_a(k, k)
      start_b(k, k)

    @pl.loop(0, BPW, step=4)
    def _(b):
      for k in range(4):
        zeros = tuple(jnp.zeros((LANES,), jnp.float32) for _ in range(DCH))
        wait_a(k)
        acc = accum(buf_a.at[k], SA, zeros)

        @pl.when(b + k + 4 < BPW)
        def _():
          start_a(b + k + 4, k)

        wait_b(k)
        acc = accum(buf_b.at[k], SB, acc)

        @pl.when(b + k + 4 < BPW)
        def _():
          start_b(b + k + 4, k)

        for c in range(DCH):
          out_v[b + k, pl.ds(c * LANES, LANES)] = acc[c] * (1.0 / S)

    pltpu.sync_copy(out_v, out_hbm.at[pl.ds(base, BPW)])

  return bag(X, table)


def _mlp_body(x_ref, w1_ref, b1_ref, w2_ref, b2_ref, o_ref):
  x = x_ref[...]
  h = jnp.maximum(
      jnp.dot(x, w1_ref[...], preferred_element_type=jnp.float32) + b1_ref[...],
      0.0,
  )
  logits = (
      jnp.dot(h, w2_ref[...], preferred_element_type=jnp.float32) + b2_ref[...]
  )
  m = jnp.max(logits, axis=-1, keepdims=True)
  s = logits - m
  lse = jnp.log(jnp.sum(jnp.exp(s), axis=-1, keepdims=True))
  o_ref[...] = s - lse


def _mlp(bag, W1, b1, W2p, b2p):
  BB = 512
  return pl.pallas_call(
      _mlp_body,
      grid=(B // BB,),
      in_specs=[
          pl.BlockSpec((BB, D), lambda i: (i, 0)),
          pl.BlockSpec((D, H), lambda i: (0, 0)),
          pl.BlockSpec((1, H), lambda i: (0, 0)),
          pl.BlockSpec((H, CPAD), lambda i: (0, 0)),
          pl.BlockSpec((1, CPAD), lambda i: (0, 0)),
      ],
      out_specs=pl.BlockSpec((BB, CPAD), lambda i: (i, 0)),
      out_shape=jax.ShapeDtypeStruct((B, CPAD), jnp.float32),
  )(bag, W1, b1, W2p, b2p)


@jax.jit
def kernel(X, table, W1, b1, W2, b2):
  Xp = jnp.pad(X, ((0, 0), (0, SP - S)))
  bag = _sc_bag(Xp, table)
  W2p = jnp.pad(W2, ((0, 0), (0, CPAD - C)))
  b2p = jnp.pad(b2, (0, CPAD - C), constant_values=-1e30).reshape(1, CPAD)
  out = _mlp(bag, W1, b1.reshape(1, H), W2p, b2p)
  return out[:, :C]
